# baseline (speedup 1.0000x reference)
import functools

import jax
import jax.numpy as jnp
import numpy as np
from jax import lax
from jax.experimental import pallas as pl
from jax.experimental.pallas import tpu as pltpu
from jax.experimental.pallas import tpu_sc as plsc

_BATCH = 4096
_CTX = 50
_D = 128
_H = 512
_C = 1000

_NC, _NS = 2, 16
_NW = _NC * _NS

_NCHUNK = 2
_NB = _BATCH // _NCHUNK
_IDX_COLS = {4096: 128, 2048: 100, 1024: 100, 512: 100}[_NB]
_BPW = _NB // _NW
_STEPS = _BPW * _CTX // _IDX_COLS
assert _BPW * _CTX % _IDX_COLS == 0 and _STEPS % 2 == 0 and _STEPS >= 4

_SEG = ((np.arange(_NS)[:, None] * _BPW)
        + (np.arange(_BPW * _CTX) // _CTX)[None, :]
        ).reshape(_NS, _STEPS, _IDX_COLS).astype(np.int32)


@functools.cache
def _make_bag_kernel():
    @functools.partial(
        pl.kernel,
        out_type=jax.ShapeDtypeStruct((_NB, _D), jnp.float32),
        mesh=plsc.VectorSubcoreMesh(core_axis_name="c", subcore_axis_name="s",
                                    num_cores=_NC, num_subcores=_NS),
        scratch_types=[
            pltpu.VMEM((_STEPS, _IDX_COLS), jnp.int32),
            pltpu.VMEM((_STEPS, _IDX_COLS), jnp.int32),
            pltpu.VMEM((_IDX_COLS, _D), jnp.float32),
            pltpu.VMEM((_IDX_COLS, _D), jnp.float32),
            pltpu.VMEM((_BPW, _D), jnp.float32),
            pltpu.VMEM_SHARED((_NS * _BPW, _D), jnp.float32),
            pltpu.SemaphoreType.DMA,
            pltpu.SemaphoreType.DMA,
        ],
    )
    def _bag_kernel(idx_hbm, seg_hbm, emb_hbm, out_hbm,
                    idx_v, seg_v, rows_a, rows_b, zero_v, acc_sh, sem_a, sem_b):
        c = lax.axis_index("c")
        s = lax.axis_index("s")
        wid = s * _NC + c
        pltpu.sync_copy(idx_hbm.at[wid], idx_v)
        pltpu.sync_copy(seg_hbm.at[s], seg_v)

        @pl.loop(0, _BPW)
        def _(i):
            @pl.loop(0, _D, step=16)
            def _(j):
                zero_v[i, pl.ds(j, 16)] = jnp.zeros((16,), jnp.float32)

        pltpu.sync_copy(zero_v, acc_sh.at[pl.ds(s * _BPW, _BPW)])

        pltpu.async_copy(emb_hbm.at[idx_v.at[0]], rows_a, sem_a)
        pltpu.async_copy(emb_hbm.at[idx_v.at[1]], rows_b, sem_b)

        @pl.loop(0, _STEPS - 2, step=2)
        def _(k):
            pltpu.make_async_copy(emb_hbm.at[idx_v.at[k]], rows_a, sem_a).wait()
            pltpu.sync_copy(rows_a, acc_sh.at[seg_v.at[k]], add=True)
            pltpu.async_copy(emb_hbm.at[idx_v.at[k + 2]], rows_a, sem_a)
            pltpu.make_async_copy(emb_hbm.at[idx_v.at[k + 1]], rows_b, sem_b).wait()
            pltpu.sync_copy(rows_b, acc_sh.at[seg_v.at[k + 1]], add=True)
            pltpu.async_copy(emb_hbm.at[idx_v.at[k + 3]], rows_b, sem_b)

        pltpu.make_async_copy(emb_hbm.at[idx_v.at[_STEPS - 2]], rows_a, sem_a).wait()
        pltpu.sync_copy(rows_a, acc_sh.at[seg_v.at[_STEPS - 2]], add=True)
        pltpu.make_async_copy(emb_hbm.at[idx_v.at[_STEPS - 1]], rows_b, sem_b).wait()
        pltpu.sync_copy(rows_b, acc_sh.at[seg_v.at[_STEPS - 1]], add=True)

        pltpu.sync_copy(acc_sh.at[pl.ds(s * _BPW, _BPW)],
                        out_hbm.at[pl.ds(wid * _BPW, _BPW)])

    return _bag_kernel


_BB = 512


def _mlp_body(bag_ref, w1_ref, b1_ref, w2_ref, b2_ref, out_ref):
    bag = bag_ref[...].astype(jnp.bfloat16)
    h = jnp.dot(bag, w1_ref[...], preferred_element_type=jnp.float32)
    h = jnp.maximum(h + b1_ref[...], 0.0).astype(jnp.bfloat16)
    logits = jnp.dot(h, w2_ref[...], preferred_element_type=jnp.float32) + b2_ref[...]
    m = jnp.max(logits, axis=-1, keepdims=True)
    lse = jnp.log(jnp.sum(jnp.exp(logits - m), axis=-1, keepdims=True)) + m
    out_ref[...] = logits - lse


_mlp = pl.pallas_call(
    _mlp_body,
    grid=(_NB // _BB,),
    in_specs=[
        pl.BlockSpec((_BB, _D), lambda i: (i, 0)),
        pl.BlockSpec((_D, _H), lambda i: (0, 0)),
        pl.BlockSpec((1, _H), lambda i: (0, 0)),
        pl.BlockSpec((_H, _C), lambda i: (0, 0)),
        pl.BlockSpec((1, _C), lambda i: (0, 0)),
    ],
    out_specs=pl.BlockSpec((_BB, _C), lambda i: (i, 0)),
    out_shape=jax.ShapeDtypeStruct((_NB, _C), jnp.float32),
)


def kernel(indices, emb, W1, b1, W2, b2):
    idx4 = indices.astype(jnp.int32).reshape(_NCHUNK, _NW, _STEPS, _IDX_COLS)
    seg = jnp.asarray(_SEG)
    w1 = W1.astype(jnp.bfloat16)
    w2 = W2.astype(jnp.bfloat16)
    b1r = b1.reshape(1, _H)
    b2r = b2.reshape(1, _C)
    bag_k = _make_bag_kernel()
    outs = []
    for i in range(_NCHUNK):
        bag_i = bag_k(idx4[i], seg, emb)
        outs.append(_mlp(bag_i, w1, b1r, w2, b2r))
    return jnp.concatenate(outs, axis=0) if _NCHUNK > 1 else outs[0]


# speedup vs baseline: 1.1188x; 1.1188x over previous
import functools

import jax
import jax.numpy as jnp
from jax import lax
from jax.experimental import pallas as pl
from jax.experimental.pallas import tpu as pltpu
from jax.experimental.pallas import tpu_sc as plsc

_BATCH = 4096
_CTX = 50
_D = 128
_H = 512
_C = 1000

_NC, _NS = 2, 16
_NW = _NC * _NS
_BPW = _BATCH // _NW


@functools.cache
def _make_bag_kernel():
    @functools.partial(
        pl.kernel,
        out_type=jax.ShapeDtypeStruct((_BATCH, _D), jnp.float32),
        mesh=plsc.VectorSubcoreMesh(core_axis_name="c", subcore_axis_name="s",
                                    num_cores=_NC, num_subcores=_NS),
        scratch_types=[
            pltpu.VMEM((_CTX, _BPW), jnp.int32),
            pltpu.VMEM((_BPW, _D), jnp.float32),
            pltpu.SemaphoreType.DMA,
        ],
    )
    def _bag_kernel(idx_hbm, emb_hbm, out_hbm, idx_v, acc_v, sem):
        c = lax.axis_index("c")
        s = lax.axis_index("s")
        wid = s * _NC + c
        pltpu.sync_copy(idx_hbm.at[wid], idx_v)

        @pl.loop(0, _BPW)
        def _(i):
            @pl.loop(0, _D, step=16)
            def _(j):
                acc_v[i, pl.ds(j, 16)] = jnp.zeros((16,), jnp.float32)

        @pl.loop(0, _CTX)
        def _(t):
            pltpu.sync_copy(emb_hbm.at[idx_v.at[t]], acc_v, add=True)

        pltpu.sync_copy(acc_v, out_hbm.at[pl.ds(wid * _BPW, _BPW)])

    return _bag_kernel


_BB = 512


def _mlp_body(bag_ref, w1_ref, b1_ref, w2_ref, b2_ref, out_ref):
    bag = bag_ref[...].astype(jnp.bfloat16)
    h = jnp.dot(bag, w1_ref[...], preferred_element_type=jnp.float32)
    h = jnp.maximum(h + b1_ref[...], 0.0).astype(jnp.bfloat16)
    logits = jnp.dot(h, w2_ref[...], preferred_element_type=jnp.float32) + b2_ref[...]
    m = jnp.max(logits, axis=-1, keepdims=True)
    lse = jnp.log(jnp.sum(jnp.exp(logits - m), axis=-1, keepdims=True)) + m
    out_ref[...] = logits - lse


_mlp = pl.pallas_call(
    _mlp_body,
    grid=(_BATCH // _BB,),
    in_specs=[
        pl.BlockSpec((_BB, _D), lambda i: (i, 0)),
        pl.BlockSpec((_D, _H), lambda i: (0, 0)),
        pl.BlockSpec((1, _H), lambda i: (0, 0)),
        pl.BlockSpec((_H, _C), lambda i: (0, 0)),
        pl.BlockSpec((1, _C), lambda i: (0, 0)),
    ],
    out_specs=pl.BlockSpec((_BB, _C), lambda i: (i, 0)),
    out_shape=jax.ShapeDtypeStruct((_BATCH, _C), jnp.float32),
)


def kernel(indices, emb, W1, b1, W2, b2):
    idx3 = indices.astype(jnp.int32).reshape(_NW, _BPW, _CTX).transpose(0, 2, 1)
    bag = _make_bag_kernel()(idx3, emb)
    return _mlp(bag, W1.astype(jnp.bfloat16), b1.reshape(1, _H),
                W2.astype(jnp.bfloat16), b2.reshape(1, _C))


# speedup vs baseline: 1.5430x; 1.3792x over previous
import functools

import jax
import jax.numpy as jnp
from jax import lax
from jax.experimental import pallas as pl
from jax.experimental.pallas import tpu as pltpu
from jax.experimental.pallas import tpu_sc as plsc

_BATCH = 4096
_CTX = 50
_D = 128
_H = 512
_C = 1000

_NC, _NS = 2, 16
_NW = _NC * _NS
_BPW = _BATCH // _NW


@functools.cache
def _make_bag_kernel():
    @functools.partial(
        pl.kernel,
        out_type=jax.ShapeDtypeStruct((_BATCH, _D), jnp.float32),
        mesh=plsc.VectorSubcoreMesh(core_axis_name="c", subcore_axis_name="s",
                                    num_cores=_NC, num_subcores=_NS),
        scratch_types=[
            pltpu.VMEM((_CTX, _BPW), jnp.int32),
            pltpu.VMEM((_BPW, _D), jnp.float32),
            pltpu.SemaphoreType.DMA,
        ],
    )
    def _bag_kernel(idx_hbm, emb_hbm, out_hbm, idx_v, acc_v, sem):
        c = lax.axis_index("c")
        s = lax.axis_index("s")
        wid = s * _NC + c
        pltpu.sync_copy(idx_hbm.at[wid], idx_v)

        @pl.loop(0, _BPW)
        def _(i):
            @pl.loop(0, _D, step=16)
            def _(j):
                acc_v[i, pl.ds(j, 16)] = jnp.zeros((16,), jnp.float32)

        @pl.loop(0, 8)
        def _(t):
            pltpu.async_copy(emb_hbm.at[idx_v.at[t]], acc_v, sem, add=True)

        @pl.loop(8, _CTX)
        def _(t):
            pltpu.make_async_copy(emb_hbm.at[idx_v.at[0]], acc_v, sem).wait()
            pltpu.async_copy(emb_hbm.at[idx_v.at[t]], acc_v, sem, add=True)

        @pl.loop(0, 8)
        def _(t):
            pltpu.make_async_copy(emb_hbm.at[idx_v.at[0]], acc_v, sem).wait()

        pltpu.sync_copy(acc_v, out_hbm.at[pl.ds(wid * _BPW, _BPW)])

    return _bag_kernel


_BB = 512


def _mlp_body(bag_ref, w1_ref, b1_ref, w2_ref, b2_ref, out_ref):
    bag = bag_ref[...].astype(jnp.bfloat16)
    h = jnp.dot(bag, w1_ref[...], preferred_element_type=jnp.float32)
    h = jnp.maximum(h + b1_ref[...], 0.0).astype(jnp.bfloat16)
    logits = jnp.dot(h, w2_ref[...], preferred_element_type=jnp.float32) + b2_ref[...]
    m = jnp.max(logits, axis=-1, keepdims=True)
    lse = jnp.log(jnp.sum(jnp.exp(logits - m), axis=-1, keepdims=True)) + m
    out_ref[...] = logits - lse


_mlp = pl.pallas_call(
    _mlp_body,
    grid=(_BATCH // _BB,),
    in_specs=[
        pl.BlockSpec((_BB, _D), lambda i: (i, 0)),
        pl.BlockSpec((_D, _H), lambda i: (0, 0)),
        pl.BlockSpec((1, _H), lambda i: (0, 0)),
        pl.BlockSpec((_H, _C), lambda i: (0, 0)),
        pl.BlockSpec((1, _C), lambda i: (0, 0)),
    ],
    out_specs=pl.BlockSpec((_BB, _C), lambda i: (i, 0)),
    out_shape=jax.ShapeDtypeStruct((_BATCH, _C), jnp.float32),
)


def kernel(indices, emb, W1, b1, W2, b2):
    idx3 = indices.astype(jnp.int32).reshape(_NW, _BPW, _CTX).transpose(0, 2, 1)
    bag = _make_bag_kernel()(idx3, emb)
    return _mlp(bag, W1.astype(jnp.bfloat16), b1.reshape(1, _H),
                W2.astype(jnp.bfloat16), b2.reshape(1, _C))
